# baseline (device time: 35271 ns/iter reference)
import jax
import jax.numpy as jnp
from jax import lax
from jax.experimental import pallas as pl
from jax.experimental.pallas import tpu as pltpu


def kernel(O, Wo):
    B, S, H, D = O.shape
    K = H * D
    N = Wo.shape[1]
    S_half = S // 2

    O2 = jnp.swapaxes(O.reshape(B, S, K), 1, 2)

    def body(o_hbm, w_hbm, out_ref, o_peer, o_mine, w_vmem,
             send_buf, recv_buf, load_sems, send_sem, recv_sem):
        my_x = lax.axis_index("x")
        my_y = lax.axis_index("y")
        my_z = lax.axis_index("z")
        peer = (1 - my_x, my_y, my_z)
        peer_start = (1 - my_x) * S_half
        my_start = my_x * S_half

        w_load = pltpu.make_async_copy(w_hbm, w_vmem, load_sems.at[2 * B])
        w_load.start()
        peer_loads, mine_loads = [], []
        for b in range(B):
            cp = pltpu.make_async_copy(
                o_hbm.at[b, :, pl.ds(peer_start, S_half)],
                o_peer.at[b], load_sems.at[b],
            )
            cp.start()
            peer_loads.append(cp)
        for b in range(B):
            cp = pltpu.make_async_copy(
                o_hbm.at[b, :, pl.ds(my_start, S_half)],
                o_mine.at[b], load_sems.at[B + b],
            )
            cp.start()
            mine_loads.append(cp)

        barrier = pltpu.get_barrier_semaphore()
        pl.semaphore_signal(
            barrier, inc=1, device_id=peer, device_id_type=pl.DeviceIdType.MESH
        )
        pl.semaphore_wait(barrier, 1)

        w_load.wait()
        w = w_vmem[...].astype(jnp.bfloat16)

        def matmul(o_half_ref, b):
            o_b = o_half_ref[b].astype(jnp.bfloat16)
            return lax.dot_general(
                o_b, w,
                dimension_numbers=(((0,), (0,)), ((), ())),
                preferred_element_type=jnp.float32,
            )

        rdmas = []
        for b in range(B):
            peer_loads[b].wait()
            send_buf[b] = matmul(o_peer, b).astype(jnp.bfloat16)
            rdma = pltpu.make_async_remote_copy(
                src_ref=send_buf.at[b],
                dst_ref=recv_buf.at[b],
                send_sem=send_sem.at[b],
                recv_sem=recv_sem.at[b],
                device_id=peer,
                device_id_type=pl.DeviceIdType.MESH,
            )
            rdma.start()
            rdmas.append(rdma)

        for b in range(B):
            mine_loads[b].wait()
            out_ref[b] = matmul(o_mine, b)

        for b in range(B):
            rdmas[b].wait()
            out_ref[b] = out_ref[b] + recv_buf[b].astype(jnp.float32)

    return pl.pallas_call(
        body,
        out_shape=jax.ShapeDtypeStruct((B, S_half, N), jnp.float32),
        in_specs=[
            pl.BlockSpec(memory_space=pl.ANY),
            pl.BlockSpec(memory_space=pl.ANY),
        ],
        out_specs=pl.BlockSpec(memory_space=pltpu.VMEM),
        scratch_shapes=[
            pltpu.VMEM((B, K, S_half), jnp.float32),
            pltpu.VMEM((B, K, S_half), jnp.float32),
            pltpu.VMEM((K, N), jnp.float32),
            pltpu.VMEM((B, S_half, N), jnp.bfloat16),
            pltpu.VMEM((B, S_half, N), jnp.bfloat16),
            pltpu.SemaphoreType.DMA((2 * B + 1,)),
            pltpu.SemaphoreType.DMA((B,)),
            pltpu.SemaphoreType.DMA((B,)),
        ],
        compiler_params=pltpu.CompilerParams(
            collective_id=0,
            vmem_limit_bytes=64 * 1024 * 1024,
        ),
    )(O2, Wo)


# device time: 33118 ns/iter; 1.0650x vs baseline; 1.0650x over previous
import jax
import jax.numpy as jnp
from jax import lax
from jax.experimental import pallas as pl
from jax.experimental.pallas import tpu as pltpu


def kernel(O, Wo):
    B, S, H, D = O.shape
    K = H * D
    N = Wo.shape[1]
    S_half = S // 2

    O2 = jnp.swapaxes(O.reshape(B, S, K), 1, 2)

    def body(o_hbm, w_hbm, out_ref, o_vmem, w_vmem, res_vmem,
             send_buf, recv_buf, load_sems, send_sem, recv_sem):
        my_x = lax.axis_index("x")
        my_y = lax.axis_index("y")
        my_z = lax.axis_index("z")
        peer = (1 - my_x, my_y, my_z)
        peer_start = (1 - my_x) * S_half
        my_start = my_x * S_half

        w_load = pltpu.make_async_copy(w_hbm, w_vmem, load_sems.at[B])
        w_load.start()
        o_loads = []
        for b in range(B):
            cp = pltpu.make_async_copy(o_hbm.at[b], o_vmem.at[b], load_sems.at[b])
            cp.start()
            o_loads.append(cp)

        barrier = pltpu.get_barrier_semaphore()
        pl.semaphore_signal(
            barrier, inc=1, device_id=peer, device_id_type=pl.DeviceIdType.MESH
        )
        pl.semaphore_wait(barrier, 1)

        w_load.wait()
        w = w_vmem[...].astype(jnp.bfloat16)

        def matmul(b, start):
            o_b = o_vmem[b, :, pl.ds(start, S_half)].astype(jnp.bfloat16)
            return lax.dot_general(
                o_b, w,
                dimension_numbers=(((0,), (0,)), ((), ())),
                preferred_element_type=jnp.float32,
            )

        rdmas = []
        for b in range(B):
            o_loads[b].wait()
            send_buf[b] = matmul(b, peer_start).astype(jnp.bfloat16)
            rdma = pltpu.make_async_remote_copy(
                src_ref=send_buf.at[b],
                dst_ref=recv_buf.at[b],
                send_sem=send_sem.at[b],
                recv_sem=recv_sem.at[b],
                device_id=peer,
                device_id_type=pl.DeviceIdType.MESH,
            )
            rdma.start()
            rdmas.append(rdma)

        for b in range(B):
            res_vmem[b] = matmul(b, my_start)

        for b in range(B):
            rdmas[b].wait()
            out_ref[b] = (
                res_vmem[b] + recv_buf[b].astype(jnp.float32)
            ).astype(jnp.bfloat16)

    return pl.pallas_call(
        body,
        out_shape=jax.ShapeDtypeStruct((B, S_half, N), jnp.bfloat16),
        in_specs=[
            pl.BlockSpec(memory_space=pl.ANY),
            pl.BlockSpec(memory_space=pl.ANY),
        ],
        out_specs=pl.BlockSpec(memory_space=pltpu.VMEM),
        scratch_shapes=[
            pltpu.VMEM((B, K, S), jnp.float32),
            pltpu.VMEM((K, N), jnp.float32),
            pltpu.VMEM((B, S_half, N), jnp.float32),
            pltpu.VMEM((B, S_half, N), jnp.bfloat16),
            pltpu.VMEM((B, S_half, N), jnp.bfloat16),
            pltpu.SemaphoreType.DMA((B + 1,)),
            pltpu.SemaphoreType.DMA((B,)),
            pltpu.SemaphoreType.DMA((B,)),
        ],
        compiler_params=pltpu.CompilerParams(
            collective_id=0,
            vmem_limit_bytes=60 * 1024 * 1024,
        ),
    )(O2, Wo)


# device time: 32553 ns/iter; 1.0835x vs baseline; 1.0174x over previous
import jax
import jax.numpy as jnp
from jax import lax
from jax.experimental import pallas as pl
from jax.experimental.pallas import tpu as pltpu


def kernel(O, Wo):
    B, S, H, D = O.shape
    K = H * D
    N = Wo.shape[1]
    S_half = S // 2
    N_half = N // 2

    O2 = jnp.swapaxes(O.reshape(B, S, K), 1, 2)

    def body(o_hbm, w_hbm, out_ref, o_vmem, w_vmem, res_vmem,
             send_buf, recv_buf, load_sems, send_sem, recv_sem):
        my_x = lax.axis_index("x")
        my_y = lax.axis_index("y")
        my_z = lax.axis_index("z")
        peer = (1 - my_x, my_y, my_z)
        peer_start = (1 - my_x) * S_half
        my_start = my_x * S_half

        w_loads = []
        for nh in range(2):
            cp = pltpu.make_async_copy(
                w_hbm.at[:, pl.ds(nh * N_half, N_half)],
                w_vmem.at[:, pl.ds(nh * N_half, N_half)],
                load_sems.at[B + nh],
            )
            w_loads.append(cp)
        o_loads = [
            pltpu.make_async_copy(o_hbm.at[b], o_vmem.at[b], load_sems.at[b])
            for b in range(B)
        ]
        w_loads[0].start()
        o_loads[0].start()

        barrier = pltpu.get_barrier_semaphore()
        pl.semaphore_signal(
            barrier, inc=1, device_id=peer, device_id_type=pl.DeviceIdType.MESH
        )
        pl.semaphore_wait(barrier, 1)

        def matmul(b, s_start, nh):
            o_b = o_vmem[b, :, pl.ds(s_start, S_half)].astype(jnp.bfloat16)
            w_nh = w_vmem[:, pl.ds(nh * N_half, N_half)].astype(jnp.bfloat16)
            return lax.dot_general(
                o_b, w_nh,
                dimension_numbers=(((0,), (0,)), ((), ())),
                preferred_element_type=jnp.float32,
            )

        rdmas = []
        for b in range(B):
            o_loads[b].wait()
            for nh in range(2):
                if b == 0:
                    w_loads[nh].wait()
                send_buf[b, :, pl.ds(nh * N_half, N_half)] = matmul(
                    b, peer_start, nh
                ).astype(jnp.bfloat16)
                rdma = pltpu.make_async_remote_copy(
                    src_ref=send_buf.at[b, :, pl.ds(nh * N_half, N_half)],
                    dst_ref=recv_buf.at[b, :, pl.ds(nh * N_half, N_half)],
                    send_sem=send_sem.at[b, nh],
                    recv_sem=recv_sem.at[b, nh],
                    device_id=peer,
                    device_id_type=pl.DeviceIdType.MESH,
                )
                rdma.start()
                rdmas.append((b, nh, rdma))
                if b == 0 and nh == 0:
                    w_loads[1].start()
                    for bb in range(1, B):
                        o_loads[bb].start()

        for b in range(B):
            res_vmem[b] = jnp.concatenate(
                [matmul(b, my_start, 0), matmul(b, my_start, 1)], axis=1
            )

        for b, nh, rdma in rdmas:
            rdma.wait()
            sl = pl.ds(nh * N_half, N_half)
            out_ref[b, :, sl] = (
                res_vmem[b, :, sl] + recv_buf[b, :, sl].astype(jnp.float32)
            ).astype(jnp.bfloat16)

    return pl.pallas_call(
        body,
        out_shape=jax.ShapeDtypeStruct((B, S_half, N), jnp.bfloat16),
        in_specs=[
            pl.BlockSpec(memory_space=pl.ANY),
            pl.BlockSpec(memory_space=pl.ANY),
        ],
        out_specs=pl.BlockSpec(memory_space=pltpu.VMEM),
        scratch_shapes=[
            pltpu.VMEM((B, K, S), jnp.float32),
            pltpu.VMEM((K, N), jnp.float32),
            pltpu.VMEM((B, S_half, N), jnp.float32),
            pltpu.VMEM((B, S_half, N), jnp.bfloat16),
            pltpu.VMEM((B, S_half, N), jnp.bfloat16),
            pltpu.SemaphoreType.DMA((B + 2,)),
            pltpu.SemaphoreType.DMA((B, 2)),
            pltpu.SemaphoreType.DMA((B, 2)),
        ],
        compiler_params=pltpu.CompilerParams(
            collective_id=0,
            vmem_limit_bytes=60 * 1024 * 1024,
        ),
    )(O2, Wo)


# device time: 26765 ns/iter; 1.3178x vs baseline; 1.2163x over previous
import jax
import jax.numpy as jnp
from jax import lax
from jax.experimental import pallas as pl
from jax.experimental.pallas import tpu as pltpu


def kernel(O, Wo):
    B, S, H, D = O.shape
    K = H * D
    N = Wo.shape[1]
    S_half = S // 2
    S_q = S_half // 2

    O2 = jnp.swapaxes(O.reshape(B, S, K), 1, 2)

    def body(o_hbm, w_hbm, out_ref, o_vmem, w_vmem, res_vmem,
             sx_buf, rx_buf, ry_buf, load_sems,
             sx_sem, rx_sem, sy_sem, ry_sem):
        my_x = lax.axis_index("x")
        my_y = lax.axis_index("y")
        my_z = lax.axis_index("z")
        x_peer = (1 - my_x, my_y, my_z)
        y_peer = (my_x, 1 - my_y, my_z)

        my_quarter = my_x * S_half + my_y * S_q
        x_peers_quarter = (1 - my_x) * S_half + my_y * S_q
        my_out_row = my_y * S_q
        other_out_row = (1 - my_y) * S_q

        w_load = pltpu.make_async_copy(w_hbm, w_vmem, load_sems.at[B])
        w_load.start()
        o_loads = []
        for b in range(B):
            cp = pltpu.make_async_copy(o_hbm.at[b], o_vmem.at[b], load_sems.at[b])
            cp.start()
            o_loads.append(cp)

        barrier = pltpu.get_barrier_semaphore()
        for nbr in (x_peer, y_peer):
            pl.semaphore_signal(
                barrier, inc=1, device_id=nbr,
                device_id_type=pl.DeviceIdType.MESH,
            )
        pl.semaphore_wait(barrier, 2)

        w_load.wait()
        w = w_vmem[...].astype(jnp.bfloat16)

        def matmul_q(b, s_start):
            o_b = o_vmem[b, :, pl.ds(s_start, S_q)].astype(jnp.bfloat16)
            return lax.dot_general(
                o_b, w,
                dimension_numbers=(((0,), (0,)), ((), ())),
                preferred_element_type=jnp.float32,
            )

        x_rdmas = []
        for b in range(B):
            o_loads[b].wait()
            sx_buf[b] = matmul_q(b, x_peers_quarter).astype(jnp.bfloat16)
            rdma = pltpu.make_async_remote_copy(
                src_ref=sx_buf.at[b],
                dst_ref=rx_buf.at[b],
                send_sem=sx_sem.at[b],
                recv_sem=rx_sem.at[b],
                device_id=x_peer,
                device_id_type=pl.DeviceIdType.MESH,
            )
            rdma.start()
            x_rdmas.append(rdma)

        for b in range(B):
            res_vmem[b] = matmul_q(b, my_quarter)

        y_rdmas = []
        for b in range(B):
            x_rdmas[b].wait()
            done = (res_vmem[b] + rx_buf[b].astype(jnp.float32)).astype(
                jnp.bfloat16
            )
            out_ref[b, pl.ds(my_out_row, S_q), :] = done
            rdma = pltpu.make_async_remote_copy(
                src_ref=out_ref.at[b, pl.ds(my_out_row, S_q), :],
                dst_ref=ry_buf.at[b],
                send_sem=sy_sem.at[b],
                recv_sem=ry_sem.at[b],
                device_id=y_peer,
                device_id_type=pl.DeviceIdType.MESH,
            )
            rdma.start()
            y_rdmas.append(rdma)

        for b in range(B):
            y_rdmas[b].wait()
            out_ref[b, pl.ds(other_out_row, S_q), :] = ry_buf[b]

    return pl.pallas_call(
        body,
        out_shape=jax.ShapeDtypeStruct((B, S_half, N), jnp.bfloat16),
        in_specs=[
            pl.BlockSpec(memory_space=pl.ANY),
            pl.BlockSpec(memory_space=pl.ANY),
        ],
        out_specs=pl.BlockSpec(memory_space=pltpu.VMEM),
        scratch_shapes=[
            pltpu.VMEM((B, K, S), jnp.float32),
            pltpu.VMEM((K, N), jnp.float32),
            pltpu.VMEM((B, S_q, N), jnp.float32),
            pltpu.VMEM((B, S_q, N), jnp.bfloat16),
            pltpu.VMEM((B, S_q, N), jnp.bfloat16),
            pltpu.VMEM((B, S_q, N), jnp.bfloat16),
            pltpu.SemaphoreType.DMA((B + 1,)),
            pltpu.SemaphoreType.DMA((B,)),
            pltpu.SemaphoreType.DMA((B,)),
            pltpu.SemaphoreType.DMA((B,)),
            pltpu.SemaphoreType.DMA((B,)),
        ],
        compiler_params=pltpu.CompilerParams(
            collective_id=0,
            vmem_limit_bytes=60 * 1024 * 1024,
        ),
    )(O2, Wo)


# device time: 23865 ns/iter; 1.4779x vs baseline; 1.1215x over previous
import jax
import jax.numpy as jnp
from jax import lax
from jax.experimental import pallas as pl
from jax.experimental.pallas import tpu as pltpu


def kernel(O, Wo):
    B, S, H, D = O.shape
    K = H * D
    N = Wo.shape[1]
    N_half = N // 2
    S_half = S // 2
    S_p = S_half // 4

    O2 = jnp.swapaxes(O.reshape(B, S, K), 1, 2)

    def body(o_hbm, w_hbm, out_ref, o_vmem, w_vmem, res_vmem,
             sx_buf, rx_buf, rz_buf, ry_buf, ryf_buf, rzf_buf,
             load_sems, x_sem, rx_sem, z_sem, rz_sem, y_sem, ry_sem,
             yf_sem, ryf_sem, zf_sem, rzf_sem):
        my_x = lax.axis_index("x")
        my_y = lax.axis_index("y")
        my_z = lax.axis_index("z")
        my_c = my_z % 2
        x_peer = (1 - my_x, my_y, my_z)
        y_peer = (my_x, 1 - my_y, my_z)
        z_pair = (my_x, my_y, my_z + 1 - 2 * my_c)

        my_quarter_src = my_x * S_half + my_y * (2 * S_p)
        x_quarter_src = (1 - my_x) * S_half + my_y * (2 * S_p)
        row_mine = my_y * (2 * S_p) + my_c * S_p
        row_from_z = my_y * (2 * S_p) + (1 - my_c) * S_p
        row_from_y = (1 - my_y) * (2 * S_p) + my_c * S_p
        row_diag = (1 - my_y) * (2 * S_p) + (1 - my_c) * S_p

        w_load = pltpu.make_async_copy(w_hbm, w_vmem, load_sems.at[B])
        w_load.start()
        o_loads = []
        for b in range(B):
            cp = pltpu.make_async_copy(o_hbm.at[b], o_vmem.at[b], load_sems.at[b])
            cp.start()
            o_loads.append(cp)

        barrier = pltpu.get_barrier_semaphore()
        for nbr in (x_peer, y_peer, z_pair):
            pl.semaphore_signal(
                barrier, inc=1, device_id=nbr,
                device_id_type=pl.DeviceIdType.MESH,
            )
        pl.semaphore_wait(barrier, 3)

        w_load.wait()
        w = w_vmem[...].astype(jnp.bfloat16)

        S_q = 2 * S_p

        def matmul_quarter(b, s_start):
            o_b = o_vmem[b, :, pl.ds(s_start, S_q)].astype(jnp.bfloat16)
            return lax.dot_general(
                o_b, w,
                dimension_numbers=(((0,), (0,)), ((), ())),
                preferred_element_type=jnp.float32,
            )

        def exchange(src_ref, dst_ref, send_sem, recv_sem, target):
            r = pltpu.make_async_remote_copy(
                src_ref=src_ref, dst_ref=dst_ref,
                send_sem=send_sem, recv_sem=recv_sem,
                device_id=target, device_id_type=pl.DeviceIdType.MESH,
            )
            r.start()
            return r

        x_rdmas = []
        for b in range(B):
            o_loads[b].wait()
            sx_buf[b] = matmul_quarter(b, x_quarter_src).astype(jnp.bfloat16)
            x_rdmas.append(exchange(
                sx_buf.at[b, pl.ds(my_c * S_p, S_p), :],
                rx_buf.at[b], x_sem.at[b], rx_sem.at[b], x_peer
            ))
        for b in range(B):
            res_vmem[b] = matmul_quarter(b, my_quarter_src)

        z_rdmas, y_rdmas = [], []
        for b in range(B):
            x_rdmas[b].wait()
            out_ref[b, pl.ds(row_mine, S_p), :] = (
                res_vmem[b, pl.ds(my_c * S_p, S_p), :]
                + rx_buf[b].astype(jnp.float32)
            ).astype(jnp.bfloat16)
            src = out_ref.at[b, pl.ds(row_mine, S_p), :]
            z_rdmas.append(exchange(
                src, rz_buf.at[b], z_sem.at[b], rz_sem.at[b], z_pair
            ))
            y_rdmas.append(exchange(
                src, ry_buf.at[b], y_sem.at[b], ry_sem.at[b], y_peer
            ))

        yf_rdmas = []
        for b in range(B):
            z_rdmas[b].wait()
            out_ref[b, pl.ds(row_from_z, S_p), :] = rz_buf[b]
            yf_rdmas.append(exchange(
                rz_buf.at[b, :, pl.ds(0, N_half)], ryf_buf.at[b],
                yf_sem.at[b], ryf_sem.at[b], y_peer
            ))

        zf_rdmas = []
        for b in range(B):
            y_rdmas[b].wait()
            out_ref[b, pl.ds(row_from_y, S_p), :] = ry_buf[b]
            zf_rdmas.append(exchange(
                ry_buf.at[b, :, pl.ds(N_half, N_half)], rzf_buf.at[b],
                zf_sem.at[b], rzf_sem.at[b], z_pair
            ))

        for b in range(B):
            yf_rdmas[b].wait()
            out_ref[b, pl.ds(row_diag, S_p), pl.ds(0, N_half)] = ryf_buf[b]
        for b in range(B):
            zf_rdmas[b].wait()
            out_ref[b, pl.ds(row_diag, S_p), pl.ds(N_half, N_half)] = rzf_buf[b]

    return pl.pallas_call(
        body,
        out_shape=jax.ShapeDtypeStruct((B, S_half, N), jnp.bfloat16),
        in_specs=[
            pl.BlockSpec(memory_space=pl.ANY),
            pl.BlockSpec(memory_space=pl.ANY),
        ],
        out_specs=pl.BlockSpec(memory_space=pltpu.VMEM),
        scratch_shapes=[
            pltpu.VMEM((B, K, S), jnp.float32),
            pltpu.VMEM((K, N), jnp.float32),
            pltpu.VMEM((B, S_half // 2, N), jnp.float32),
            pltpu.VMEM((B, S_half // 2, N), jnp.bfloat16),
            pltpu.VMEM((B, S_p, N), jnp.bfloat16),
            pltpu.VMEM((B, S_p, N), jnp.bfloat16),
            pltpu.VMEM((B, S_p, N), jnp.bfloat16),
            pltpu.VMEM((B, S_p, N_half), jnp.bfloat16),
            pltpu.VMEM((B, S_p, N_half), jnp.bfloat16),
            pltpu.SemaphoreType.DMA((B + 1,)),
            pltpu.SemaphoreType.DMA((B,)),
            pltpu.SemaphoreType.DMA((B,)),
            pltpu.SemaphoreType.DMA((B,)),
            pltpu.SemaphoreType.DMA((B,)),
            pltpu.SemaphoreType.DMA((B,)),
            pltpu.SemaphoreType.DMA((B,)),
            pltpu.SemaphoreType.DMA((B,)),
            pltpu.SemaphoreType.DMA((B,)),
            pltpu.SemaphoreType.DMA((B,)),
            pltpu.SemaphoreType.DMA((B,)),
        ],
        compiler_params=pltpu.CompilerParams(
            collective_id=0,
            vmem_limit_bytes=60 * 1024 * 1024,
        ),
    )(O2, Wo)


# device time: 23547 ns/iter; 1.4979x vs baseline; 1.0135x over previous
import jax
import jax.numpy as jnp
from jax import lax
from jax.experimental import pallas as pl
from jax.experimental.pallas import tpu as pltpu


def kernel(O, Wo):
    B, S, H, D = O.shape
    K = H * D
    N = Wo.shape[1]
    N_half = N // 2
    S_half = S // 2
    S_p = S_half // 4

    O2 = jnp.swapaxes(O.reshape(B, S, K), 1, 2)

    def body(o_hbm, w_hbm, out_ref, o_vmem, w_vmem, res_vmem,
             sx_buf, rx_buf, rz_buf, ry_buf, ryf_buf, rzf_buf,
             load_sems, x_sem, rx_sem, z_sem, rz_sem, y_sem, ry_sem,
             yf_sem, ryf_sem, zf_sem, rzf_sem):
        my_x = lax.axis_index("x")
        my_y = lax.axis_index("y")
        my_z = lax.axis_index("z")
        my_c = my_z % 2
        x_peer = (1 - my_x, my_y, my_z)
        y_peer = (my_x, 1 - my_y, my_z)
        z_pair = (my_x, my_y, my_z + 1 - 2 * my_c)

        my_quarter_src = my_x * S_half + my_y * (2 * S_p)
        x_quarter_src = (1 - my_x) * S_half + my_y * (2 * S_p)
        row_mine = my_y * (2 * S_p) + my_c * S_p
        row_from_z = my_y * (2 * S_p) + (1 - my_c) * S_p
        row_from_y = (1 - my_y) * (2 * S_p) + my_c * S_p
        row_diag = (1 - my_y) * (2 * S_p) + (1 - my_c) * S_p

        w_loads = []
        for nh in range(2):
            w_loads.append(pltpu.make_async_copy(
                w_hbm.at[:, pl.ds(nh * N_half, N_half)],
                w_vmem.at[:, pl.ds(nh * N_half, N_half)],
                load_sems.at[B + nh],
            ))
        o_loads = [
            pltpu.make_async_copy(o_hbm.at[b], o_vmem.at[b], load_sems.at[b])
            for b in range(B)
        ]
        w_loads[0].start()
        o_loads[0].start()
        w_loads[1].start()
        for b in range(1, B):
            o_loads[b].start()

        barrier = pltpu.get_barrier_semaphore()
        for nbr in (x_peer, y_peer, z_pair):
            pl.semaphore_signal(
                barrier, inc=1, device_id=nbr,
                device_id_type=pl.DeviceIdType.MESH,
            )
        pl.semaphore_wait(barrier, 3)

        S_q = 2 * S_p

        def matmul_quarter(b, s_start, nh=None):
            o_b = o_vmem[b, :, pl.ds(s_start, S_q)].astype(jnp.bfloat16)
            if nh is None:
                w_op = w_vmem[...]
            else:
                w_op = w_vmem[:, pl.ds(nh * N_half, N_half)]
            return lax.dot_general(
                o_b, w_op.astype(jnp.bfloat16),
                dimension_numbers=(((0,), (0,)), ((), ())),
                preferred_element_type=jnp.float32,
            )

        def exchange(src_ref, dst_ref, send_sem, recv_sem, target):
            r = pltpu.make_async_remote_copy(
                src_ref=src_ref, dst_ref=dst_ref,
                send_sem=send_sem, recv_sem=recv_sem,
                device_id=target, device_id_type=pl.DeviceIdType.MESH,
            )
            r.start()
            return r

        x_rdmas = []
        for b in range(B):
            o_loads[b].wait()
            for nh in range(2):
                if b == 0:
                    w_loads[nh].wait()
                nsl = pl.ds(nh * N_half, N_half)
                sx_buf[b, :, nsl] = matmul_quarter(
                    b, x_quarter_src, nh
                ).astype(jnp.bfloat16)
                x_rdmas.append(exchange(
                    sx_buf.at[b, pl.ds(my_c * S_p, S_p), nsl],
                    rx_buf.at[b, :, nsl],
                    x_sem.at[b, nh], rx_sem.at[b, nh], x_peer
                ))
        for b in range(B):
            res_vmem[b] = matmul_quarter(b, my_quarter_src)

        z_rdmas, y_rdmas = [], []
        for b in range(B):
            x_rdmas[2 * b].wait()
            x_rdmas[2 * b + 1].wait()
            out_ref[b, pl.ds(row_mine, S_p), :] = (
                res_vmem[b, pl.ds(my_c * S_p, S_p), :]
                + rx_buf[b].astype(jnp.float32)
            ).astype(jnp.bfloat16)
            src = out_ref.at[b, pl.ds(row_mine, S_p), :]
            z_rdmas.append(exchange(
                src, rz_buf.at[b], z_sem.at[b], rz_sem.at[b], z_pair
            ))
            y_rdmas.append(exchange(
                src, ry_buf.at[b], y_sem.at[b], ry_sem.at[b], y_peer
            ))

        yf_rdmas = []
        for b in range(B):
            z_rdmas[b].wait()
            out_ref[b, pl.ds(row_from_z, S_p), :] = rz_buf[b]
            yf_rdmas.append(exchange(
                rz_buf.at[b, :, pl.ds(0, N_half)], ryf_buf.at[b],
                yf_sem.at[b], ryf_sem.at[b], y_peer
            ))

        zf_rdmas = []
        for b in range(B):
            y_rdmas[b].wait()
            out_ref[b, pl.ds(row_from_y, S_p), :] = ry_buf[b]
            zf_rdmas.append(exchange(
                ry_buf.at[b, :, pl.ds(N_half, N_half)], rzf_buf.at[b],
                zf_sem.at[b], rzf_sem.at[b], z_pair
            ))

        for b in range(B):
            yf_rdmas[b].wait()
            out_ref[b, pl.ds(row_diag, S_p), pl.ds(0, N_half)] = ryf_buf[b]
        for b in range(B):
            zf_rdmas[b].wait()
            out_ref[b, pl.ds(row_diag, S_p), pl.ds(N_half, N_half)] = rzf_buf[b]

    return pl.pallas_call(
        body,
        out_shape=jax.ShapeDtypeStruct((B, S_half, N), jnp.bfloat16),
        in_specs=[
            pl.BlockSpec(memory_space=pl.ANY),
            pl.BlockSpec(memory_space=pl.ANY),
        ],
        out_specs=pl.BlockSpec(memory_space=pltpu.VMEM),
        scratch_shapes=[
            pltpu.VMEM((B, K, S), jnp.float32),
            pltpu.VMEM((K, N), jnp.float32),
            pltpu.VMEM((B, S_half // 2, N), jnp.float32),
            pltpu.VMEM((B, S_half // 2, N), jnp.bfloat16),
            pltpu.VMEM((B, S_p, N), jnp.bfloat16),
            pltpu.VMEM((B, S_p, N), jnp.bfloat16),
            pltpu.VMEM((B, S_p, N), jnp.bfloat16),
            pltpu.VMEM((B, S_p, N_half), jnp.bfloat16),
            pltpu.VMEM((B, S_p, N_half), jnp.bfloat16),
            pltpu.SemaphoreType.DMA((B + 2,)),
            pltpu.SemaphoreType.DMA((B, 2)),
            pltpu.SemaphoreType.DMA((B, 2)),
            pltpu.SemaphoreType.DMA((B,)),
            pltpu.SemaphoreType.DMA((B,)),
            pltpu.SemaphoreType.DMA((B,)),
            pltpu.SemaphoreType.DMA((B,)),
            pltpu.SemaphoreType.DMA((B,)),
            pltpu.SemaphoreType.DMA((B,)),
            pltpu.SemaphoreType.DMA((B,)),
            pltpu.SemaphoreType.DMA((B,)),
        ],
        compiler_params=pltpu.CompilerParams(
            collective_id=0,
            vmem_limit_bytes=60 * 1024 * 1024,
        ),
    )(O2, Wo)
